# baseline (speedup 1.0000x reference)
import functools

import jax
import jax.numpy as jnp
from jax import lax
from jax.experimental import pallas as pl
from jax.experimental.pallas import tpu as pltpu
from jax.experimental.pallas import tpu_sc as plsc

N_NODES = 10000
N_EDGES = 320000
D = 128
DH = D // 2
NUM_CLASSES = 7

NC = 2
NS = 16
EPT = N_EDGES // NS
K = 80
NCHUNK = EPT // K
NB = 5
ROWS_PER_TILE = N_NODES // NS


@functools.cache
def _build_sc_aggregate():
    mesh = plsc.VectorSubcoreMesh(core_axis_name="c", subcore_axis_name="s",
                                  num_cores=NC, num_subcores=NS)
    return functools.partial(
        pl.kernel,
        mesh=mesh,
        out_type=jax.ShapeDtypeStruct((NC, N_NODES, DH), jnp.float32),
        scratch_types=[
            pltpu.VMEM((NCHUNK, K), jnp.int32),
            pltpu.VMEM((NCHUNK, K), jnp.int32),
            pltpu.VMEM((NB, K, DH), jnp.float32),
            pltpu.VMEM_SHARED((N_NODES, DH), jnp.float32),
            pltpu.SemaphoreType.DMA,
            pltpu.SemaphoreType.DMA,
            pltpu.SemaphoreType.DMA,
            pltpu.SemaphoreType.DMA,
            pltpu.SemaphoreType.DMA,
            pltpu.SemaphoreType.DMA,
            pltpu.SemaphoreType.DMA,
            pltpu.SemaphoreType.DMA,
            pltpu.SemaphoreType.DMA,
            pltpu.SemaphoreType.DMA,
        ],
        compiler_params=pltpu.CompilerParams(use_tc_tiling_on_sc=False),
    )(_sc_aggregate_body)


def _sc_aggregate_body(hh_hbm, src_hbm, dst_hbm, zeros_hbm, out_hbm,
                       src_v, dst_v, rows_v, agg_sh,
                       gsem0, gsem1, gsem2, gsem3, gsem4,
                       ssem0, ssem1, ssem2, ssem3, ssem4):
    c = lax.axis_index("c")
    s = lax.axis_index("s")
    gsems = (gsem0, gsem1, gsem2, gsem3, gsem4)
    ssems = (ssem0, ssem1, ssem2, ssem3, ssem4)

    h_hbm = hh_hbm.at[c]

    pltpu.sync_copy(src_hbm.at[s], src_v)
    pltpu.sync_copy(dst_hbm.at[s], dst_v)
    pltpu.sync_copy(zeros_hbm,
                    agg_sh.at[pl.ds(s * ROWS_PER_TILE, ROWS_PER_TILE)])
    plsc.subcore_barrier()

    LEAD = 3
    for b in range(LEAD):
        pltpu.async_copy(h_hbm.at[src_v.at[b]], rows_v.at[b], gsems[b])

    def body(jj, carry):
        del carry
        for b in range(NB):
            j = NB * jj + b
            nb = (b + LEAD) % NB
            pltpu.make_async_copy(h_hbm.at[src_v.at[j]], rows_v.at[b],
                                  gsems[b]).wait()
            pltpu.async_copy(rows_v.at[b], agg_sh.at[dst_v.at[j]],
                             ssems[b], add=True)

            @pl.when(jnp.logical_and(j >= NB - LEAD, j + LEAD < NCHUNK))
            def _():
                pltpu.make_async_copy(rows_v.at[nb], agg_sh.at[dst_v.at[nb]],
                                      ssems[nb]).wait()
                pltpu.async_copy(h_hbm.at[src_v.at[j + LEAD]], rows_v.at[nb],
                                 gsems[nb])

            @pl.when(j < NB - LEAD)
            def _():
                pltpu.async_copy(h_hbm.at[src_v.at[j + LEAD]], rows_v.at[nb],
                                 gsems[nb])
        return 0

    lax.fori_loop(0, NCHUNK // NB, body, 0)
    for b in range(NB):
        pltpu.make_async_copy(rows_v.at[b], agg_sh.at[dst_v.at[b]],
                              ssems[b]).wait()

    plsc.subcore_barrier()
    pltpu.sync_copy(agg_sh.at[pl.ds(s * 624, 624)],
                    out_hbm.at[c, pl.ds(s * 624, 624)])

    @pl.when(s == NS - 1)
    def _():
        pltpu.sync_copy(agg_sh.at[pl.ds(16 * 624, N_NODES - 16 * 624)],
                        out_hbm.at[c, pl.ds(16 * 624, N_NODES - 16 * 624)])


def _mlp_body(h_ref, a0_ref, a1_ref, w1_ref, b1_ref, w2_ref, b2_ref,
              out_ref, outh_ref):
    z = h_ref[...] + jnp.concatenate((a0_ref[...], a1_ref[...]), axis=1)
    t = jnp.dot(z, w1_ref[...], preferred_element_type=jnp.float32) + b1_ref[...]
    t = jnp.maximum(t, 0.0)
    o = jnp.dot(t, w2_ref[...], preferred_element_type=jnp.float32) + b2_ref[...]
    o = jnp.maximum(o, 0.0)
    out_ref[...] = o
    outh_ref[0] = o[:, :DH]
    outh_ref[1] = o[:, DH:]


def _head_body(h_ref, a0_ref, a1_ref, w1_ref, b1_ref, w2_ref, b2_ref,
               wc1_ref, bc1_ref, wc2_ref, bc2_ref, out_ref):
    z = h_ref[...] + jnp.concatenate((a0_ref[...], a1_ref[...]), axis=1)
    t = jnp.dot(z, w1_ref[...], preferred_element_type=jnp.float32) + b1_ref[...]
    t = jnp.maximum(t, 0.0)
    h2 = jnp.dot(t, w2_ref[...], preferred_element_type=jnp.float32) + b2_ref[...]
    h2 = jnp.maximum(h2, 0.0)
    hc = jnp.dot(h2, wc1_ref[...], preferred_element_type=jnp.float32) + bc1_ref[...]
    hc = jnp.maximum(hc, 0.0)
    out_ref[...] = jnp.dot(hc, wc2_ref[...], preferred_element_type=jnp.float32) + bc2_ref[...]


_ROW_BLK = 1000


def _row_spec():
    return pl.BlockSpec((_ROW_BLK, D), lambda i: (i, 0))


def _half_spec():
    return pl.BlockSpec((_ROW_BLK, DH), lambda i: (i, 0))


def _full_spec():
    return pl.BlockSpec((D, D), lambda i: (0, 0))


def _bias_spec():
    return pl.BlockSpec((1, D), lambda i: (0, 0))


def _tc_mlp(h, a0, a1, w1, b1, w2, b2):
    return pl.pallas_call(
        _mlp_body,
        grid=(N_NODES // _ROW_BLK,),
        in_specs=[_row_spec(), _half_spec(), _half_spec(),
                  _full_spec(), _bias_spec(), _full_spec(), _bias_spec()],
        out_specs=[_row_spec(),
                   pl.BlockSpec((2, _ROW_BLK, DH), lambda i: (0, i, 0))],
        out_shape=[jax.ShapeDtypeStruct((N_NODES, D), jnp.float32),
                   jax.ShapeDtypeStruct((2, N_NODES, DH), jnp.float32)],
    )(h, a0, a1, w1, b1.reshape(1, D), w2, b2.reshape(1, D))


def _tc_head(h, a0, a1, w1, b1, w2, b2, wc1, bc1, wc2p, bc2p):
    return pl.pallas_call(
        _head_body,
        grid=(N_NODES // _ROW_BLK,),
        in_specs=[_row_spec(), _half_spec(), _half_spec(),
                  _full_spec(), _bias_spec(), _full_spec(), _bias_spec(),
                  _full_spec(), _bias_spec(), _full_spec(), _bias_spec()],
        out_specs=_row_spec(),
        out_shape=jax.ShapeDtypeStruct((N_NODES, D), jnp.float32),
    )(h, a0, a1, w1, b1.reshape(1, D), w2, b2.reshape(1, D),
      wc1, bc1.reshape(1, D), wc2p, bc2p)


def kernel(x, edge_index, batch,
           W1_0, b1_0, W2_0, b2_0,
           W1_1, b1_1, W2_1, b2_1,
           Wc1, bc1, Wc2, bc2):
    del batch
    src = edge_index[0].astype(jnp.int32).reshape(NS, NCHUNK, K)
    dst = edge_index[1].astype(jnp.int32).reshape(NS, NCHUNK, K)
    zeros = jnp.zeros((ROWS_PER_TILE, DH), jnp.float32)

    wc2p = jnp.zeros((D, D), jnp.float32).at[:, :NUM_CLASSES].set(Wc2)
    bc2p = jnp.zeros((1, D), jnp.float32).at[0, :NUM_CLASSES].set(bc2)

    sc_aggregate = _build_sc_aggregate()
    xh = x.reshape(N_NODES, NC, DH).transpose(1, 0, 2)
    agg = sc_aggregate(xh, src, dst, zeros)
    h1, h1h = _tc_mlp(x, agg[0], agg[1], W1_0, b1_0, W2_0, b2_0)
    agg = sc_aggregate(h1h, src, dst, zeros)
    logits_p = _tc_head(h1, agg[0], agg[1], W1_1, b1_1, W2_1, b2_1,
                        Wc1, bc1, wc2p, bc2p)
    logits = logits_p[:, :NUM_CLASSES]
    return (logits, jnp.float32(0.0), jnp.float32(0.0))


# speedup vs baseline: 1.0806x; 1.0806x over previous
import functools

import jax
import jax.numpy as jnp
from jax import lax
from jax.experimental import pallas as pl
from jax.experimental.pallas import tpu as pltpu
from jax.experimental.pallas import tpu_sc as plsc

N_NODES = 10000
N_EDGES = 320000
D = 128
DH = D // 2
NUM_CLASSES = 7

NC = 2
NS = 16
EPT = N_EDGES // NS
K = 80
NCHUNK = EPT // K
NB = 5
ROWS_PER_TILE = N_NODES // NS


@functools.cache
def _build_sc_aggregate():
    mesh = plsc.VectorSubcoreMesh(core_axis_name="c", subcore_axis_name="s",
                                  num_cores=NC, num_subcores=NS)
    return functools.partial(
        pl.kernel,
        mesh=mesh,
        out_type=jax.ShapeDtypeStruct((NC, N_NODES, DH), jnp.float32),
        scratch_types=[
            pltpu.VMEM((NCHUNK, K), jnp.int32),
            pltpu.VMEM((NCHUNK, K), jnp.int32),
            pltpu.VMEM((NB, K, DH), jnp.float32),
            pltpu.VMEM_SHARED((N_NODES, DH), jnp.float32),
            pltpu.SemaphoreType.DMA,
            pltpu.SemaphoreType.DMA,
            pltpu.SemaphoreType.DMA,
            pltpu.SemaphoreType.DMA,
            pltpu.SemaphoreType.DMA,
            pltpu.SemaphoreType.DMA,
            pltpu.SemaphoreType.DMA,
            pltpu.SemaphoreType.DMA,
            pltpu.SemaphoreType.DMA,
            pltpu.SemaphoreType.DMA,
        ],
        compiler_params=pltpu.CompilerParams(use_tc_tiling_on_sc=False),
    )(_sc_aggregate_body)


def _sc_aggregate_body(hh_hbm, src_hbm, dst_hbm, zeros_hbm, out_hbm,
                       src_v, dst_v, rows_v, agg_sh,
                       gsem0, gsem1, gsem2, gsem3, gsem4,
                       ssem0, ssem1, ssem2, ssem3, ssem4):
    c = lax.axis_index("c")
    s = lax.axis_index("s")
    gsems = (gsem0, gsem1, gsem2, gsem3, gsem4)
    ssems = (ssem0, ssem1, ssem2, ssem3, ssem4)

    h_hbm = hh_hbm.at[c]

    pltpu.sync_copy(src_hbm.at[s], src_v)
    pltpu.sync_copy(dst_hbm.at[s], dst_v)
    pltpu.sync_copy(zeros_hbm,
                    agg_sh.at[pl.ds(s * ROWS_PER_TILE, ROWS_PER_TILE)])
    plsc.subcore_barrier()

    for b in range(NB):
        pltpu.async_copy(h_hbm.at[src_v.at[b]], rows_v.at[b], gsems[b])

    def body(jj, carry):
        del carry
        for b in range(NB):
            j = NB * jj + b
            pltpu.make_async_copy(h_hbm.at[src_v.at[j]], rows_v.at[b],
                                  gsems[b]).wait()
            cp = pltpu.async_copy(rows_v.at[b], agg_sh.at[dst_v.at[j]],
                                  ssems[b], add=True)
            cp.wait()

            @pl.when(j + NB < NCHUNK)
            def _():
                pltpu.async_copy(h_hbm.at[src_v.at[j + NB]], rows_v.at[b],
                                 gsems[b])
        return 0

    lax.fori_loop(0, NCHUNK // NB, body, 0)

    plsc.subcore_barrier()
    pltpu.sync_copy(agg_sh.at[pl.ds(s * 624, 624)],
                    out_hbm.at[c, pl.ds(s * 624, 624)])

    @pl.when(s == NS - 1)
    def _():
        pltpu.sync_copy(agg_sh.at[pl.ds(16 * 624, N_NODES - 16 * 624)],
                        out_hbm.at[c, pl.ds(16 * 624, N_NODES - 16 * 624)])


def _mlp_body(h_ref, a0_ref, a1_ref, w1_ref, b1_ref, w2_ref, b2_ref,
              out_ref, outh_ref):
    z = h_ref[...] + jnp.concatenate((a0_ref[...], a1_ref[...]), axis=1)
    t = jnp.dot(z, w1_ref[...], preferred_element_type=jnp.float32) + b1_ref[...]
    t = jnp.maximum(t, 0.0)
    o = jnp.dot(t, w2_ref[...], preferred_element_type=jnp.float32) + b2_ref[...]
    o = jnp.maximum(o, 0.0)
    out_ref[...] = o
    outh_ref[0] = o[:, :DH]
    outh_ref[1] = o[:, DH:]


def _head_body(h_ref, a0_ref, a1_ref, w1_ref, b1_ref, w2_ref, b2_ref,
               wc1_ref, bc1_ref, wc2_ref, bc2_ref, out_ref):
    z = h_ref[...] + jnp.concatenate((a0_ref[...], a1_ref[...]), axis=1)
    t = jnp.dot(z, w1_ref[...], preferred_element_type=jnp.float32) + b1_ref[...]
    t = jnp.maximum(t, 0.0)
    h2 = jnp.dot(t, w2_ref[...], preferred_element_type=jnp.float32) + b2_ref[...]
    h2 = jnp.maximum(h2, 0.0)
    hc = jnp.dot(h2, wc1_ref[...], preferred_element_type=jnp.float32) + bc1_ref[...]
    hc = jnp.maximum(hc, 0.0)
    out_ref[...] = jnp.dot(hc, wc2_ref[...], preferred_element_type=jnp.float32) + bc2_ref[...]


_ROW_BLK = 1000


def _row_spec():
    return pl.BlockSpec((_ROW_BLK, D), lambda i: (i, 0))


def _half_spec():
    return pl.BlockSpec((_ROW_BLK, DH), lambda i: (i, 0))


def _full_spec():
    return pl.BlockSpec((D, D), lambda i: (0, 0))


def _bias_spec():
    return pl.BlockSpec((1, D), lambda i: (0, 0))


def _tc_mlp(h, a0, a1, w1, b1, w2, b2):
    return pl.pallas_call(
        _mlp_body,
        grid=(N_NODES // _ROW_BLK,),
        in_specs=[_row_spec(), _half_spec(), _half_spec(),
                  _full_spec(), _bias_spec(), _full_spec(), _bias_spec()],
        out_specs=[_row_spec(),
                   pl.BlockSpec((2, _ROW_BLK, DH), lambda i: (0, i, 0))],
        out_shape=[jax.ShapeDtypeStruct((N_NODES, D), jnp.float32),
                   jax.ShapeDtypeStruct((2, N_NODES, DH), jnp.float32)],
    )(h, a0, a1, w1, b1.reshape(1, D), w2, b2.reshape(1, D))


def _tc_head(h, a0, a1, w1, b1, w2, b2, wc1, bc1, wc2p, bc2p):
    return pl.pallas_call(
        _head_body,
        grid=(N_NODES // _ROW_BLK,),
        in_specs=[_row_spec(), _half_spec(), _half_spec(),
                  _full_spec(), _bias_spec(), _full_spec(), _bias_spec(),
                  _full_spec(), _bias_spec(), _full_spec(), _bias_spec()],
        out_specs=_row_spec(),
        out_shape=jax.ShapeDtypeStruct((N_NODES, D), jnp.float32),
    )(h, a0, a1, w1, b1.reshape(1, D), w2, b2.reshape(1, D),
      wc1, bc1.reshape(1, D), wc2p, bc2p)


def kernel(x, edge_index, batch,
           W1_0, b1_0, W2_0, b2_0,
           W1_1, b1_1, W2_1, b2_1,
           Wc1, bc1, Wc2, bc2):
    del batch
    src = edge_index[0].astype(jnp.int32).reshape(NS, NCHUNK, K)
    dst = edge_index[1].astype(jnp.int32).reshape(NS, NCHUNK, K)
    zeros = jnp.zeros((ROWS_PER_TILE, DH), jnp.float32)

    wc2p = jnp.zeros((D, D), jnp.float32).at[:, :NUM_CLASSES].set(Wc2)
    bc2p = jnp.zeros((1, D), jnp.float32).at[0, :NUM_CLASSES].set(bc2)

    sc_aggregate = _build_sc_aggregate()
    xh = x.reshape(N_NODES, NC, DH).transpose(1, 0, 2)
    agg = sc_aggregate(xh, src, dst, zeros)
    h1, h1h = _tc_mlp(x, agg[0], agg[1], W1_0, b1_0, W2_0, b2_0)
    agg = sc_aggregate(h1h, src, dst, zeros)
    logits_p = _tc_head(h1, agg[0], agg[1], W1_1, b1_1, W2_1, b2_1,
                        Wc1, bc1, wc2p, bc2p)
    logits = logits_p[:, :NUM_CLASSES]
    return (logits, jnp.float32(0.0), jnp.float32(0.0))


# speedup vs baseline: 1.1034x; 1.0212x over previous
import functools

import jax
import jax.numpy as jnp
from jax import lax
from jax.experimental import pallas as pl
from jax.experimental.pallas import tpu as pltpu
from jax.experimental.pallas import tpu_sc as plsc

N_NODES = 10000
N_EDGES = 320000
D = 128
DH = D // 2
NUM_CLASSES = 7

NC = 2
NS = 16
EPT = N_EDGES // NS
K = 80
NCHUNK = EPT // K
NB = 5
ROWS_PER_TILE = N_NODES // NS


@functools.cache
def _build_sc_aggregate():
    mesh = plsc.VectorSubcoreMesh(core_axis_name="c", subcore_axis_name="s",
                                  num_cores=NC, num_subcores=NS)
    return functools.partial(
        pl.kernel,
        mesh=mesh,
        out_type=jax.ShapeDtypeStruct((NC, N_NODES, DH), jnp.float32),
        scratch_types=[
            pltpu.VMEM((NCHUNK, K), jnp.int32),
            pltpu.VMEM((NCHUNK, K), jnp.int32),
            pltpu.VMEM((NB, K, DH), jnp.float32),
            pltpu.VMEM_SHARED((N_NODES, DH), jnp.float32),
            pltpu.SemaphoreType.DMA,
            pltpu.SemaphoreType.DMA,
            pltpu.SemaphoreType.DMA,
            pltpu.SemaphoreType.DMA,
            pltpu.SemaphoreType.DMA,
            pltpu.SemaphoreType.DMA,
            pltpu.SemaphoreType.DMA,
            pltpu.SemaphoreType.DMA,
            pltpu.SemaphoreType.DMA,
            pltpu.SemaphoreType.DMA,
        ],
        compiler_params=pltpu.CompilerParams(use_tc_tiling_on_sc=False),
    )(_sc_aggregate_body)


def _sc_aggregate_body(hh_hbm, src_hbm, dst_hbm, zeros_hbm, out_hbm,
                       src_v, dst_v, rows_v, agg_sh,
                       gsem0, gsem1, gsem2, gsem3, gsem4,
                       ssem0, ssem1, ssem2, ssem3, ssem4):
    c = lax.axis_index("c")
    s = lax.axis_index("s")
    gsems = (gsem0, gsem1, gsem2, gsem3, gsem4)
    ssems = (ssem0, ssem1, ssem2, ssem3, ssem4)

    h_hbm = hh_hbm.at[c]

    pltpu.sync_copy(src_hbm.at[s], src_v)
    pltpu.sync_copy(dst_hbm.at[s], dst_v)
    pltpu.sync_copy(zeros_hbm,
                    agg_sh.at[pl.ds(s * ROWS_PER_TILE, ROWS_PER_TILE)])
    plsc.subcore_barrier()

    for b in range(NB):
        pltpu.async_copy(h_hbm.at[src_v.at[b]], rows_v.at[b], gsems[b])

    def body(jj, carry):
        del carry
        for b in range(NB):
            j = NB * jj + b
            pltpu.make_async_copy(h_hbm.at[src_v.at[j]], rows_v.at[b],
                                  gsems[b]).wait()
            cp = pltpu.async_copy(rows_v.at[b], agg_sh.at[dst_v.at[j]],
                                  ssems[b], add=True)
            cp.wait()

            @pl.when(j + NB < NCHUNK)
            def _():
                pltpu.async_copy(h_hbm.at[src_v.at[j + NB]], rows_v.at[b],
                                 gsems[b])
        return 0

    lax.fori_loop(0, NCHUNK // NB, body, 0)

    plsc.subcore_barrier()
    pltpu.sync_copy(agg_sh.at[pl.ds(s * 624, 624)],
                    out_hbm.at[c, pl.ds(s * 624, 624)])

    @pl.when(s == NS - 1)
    def _():
        pltpu.sync_copy(agg_sh.at[pl.ds(16 * 624, N_NODES - 16 * 624)],
                        out_hbm.at[c, pl.ds(16 * 624, N_NODES - 16 * 624)])


def _mlp_body(h_ref, a0_ref, a1_ref, w1_ref, b1_ref, w2_ref, b2_ref,
              out_ref, outh_ref):
    z = h_ref[...] + jnp.concatenate((a0_ref[...], a1_ref[...]), axis=1)
    t = jnp.dot(z, w1_ref[...], preferred_element_type=jnp.float32) + b1_ref[...]
    t = jnp.maximum(t, 0.0)
    o = jnp.dot(t, w2_ref[...], preferred_element_type=jnp.float32) + b2_ref[...]
    o = jnp.maximum(o, 0.0)
    out_ref[...] = o
    outh_ref[0] = o[:, :DH]
    outh_ref[1] = o[:, DH:]


def _head_body(h_ref, a0_ref, a1_ref, w1_ref, b1_ref, w2_ref, b2_ref,
               wc1_ref, bc1_ref, wc2_ref, bc2_ref, out_ref):
    z = h_ref[...] + jnp.concatenate((a0_ref[...], a1_ref[...]), axis=1)
    t = jnp.dot(z, w1_ref[...], preferred_element_type=jnp.float32) + b1_ref[...]
    t = jnp.maximum(t, 0.0)
    h2 = jnp.dot(t, w2_ref[...], preferred_element_type=jnp.float32) + b2_ref[...]
    h2 = jnp.maximum(h2, 0.0)
    hc = jnp.dot(h2, wc1_ref[...], preferred_element_type=jnp.float32) + bc1_ref[...]
    hc = jnp.maximum(hc, 0.0)
    out_ref[...] = jnp.dot(hc, wc2_ref[...], preferred_element_type=jnp.float32) + bc2_ref[...]


_ROW_BLK = 10000


def _row_spec():
    return pl.BlockSpec((_ROW_BLK, D), lambda i: (i, 0))


def _half_spec():
    return pl.BlockSpec((_ROW_BLK, DH), lambda i: (i, 0))


def _full_spec():
    return pl.BlockSpec((D, D), lambda i: (0, 0))


def _bias_spec():
    return pl.BlockSpec((1, D), lambda i: (0, 0))


def _tc_mlp(h, a0, a1, w1, b1, w2, b2):
    return pl.pallas_call(
        _mlp_body,
        grid=(N_NODES // _ROW_BLK,),
        in_specs=[_row_spec(), _half_spec(), _half_spec(),
                  _full_spec(), _bias_spec(), _full_spec(), _bias_spec()],
        out_specs=[_row_spec(),
                   pl.BlockSpec((2, _ROW_BLK, DH), lambda i: (0, i, 0))],
        out_shape=[jax.ShapeDtypeStruct((N_NODES, D), jnp.float32),
                   jax.ShapeDtypeStruct((2, N_NODES, DH), jnp.float32)],
    )(h, a0, a1, w1, b1.reshape(1, D), w2, b2.reshape(1, D))


def _tc_head(h, a0, a1, w1, b1, w2, b2, wc1, bc1, wc2p, bc2p):
    return pl.pallas_call(
        _head_body,
        grid=(N_NODES // _ROW_BLK,),
        in_specs=[_row_spec(), _half_spec(), _half_spec(),
                  _full_spec(), _bias_spec(), _full_spec(), _bias_spec(),
                  _full_spec(), _bias_spec(), _full_spec(), _bias_spec()],
        out_specs=_row_spec(),
        out_shape=jax.ShapeDtypeStruct((N_NODES, D), jnp.float32),
    )(h, a0, a1, w1, b1.reshape(1, D), w2, b2.reshape(1, D),
      wc1, bc1.reshape(1, D), wc2p, bc2p)


def kernel(x, edge_index, batch,
           W1_0, b1_0, W2_0, b2_0,
           W1_1, b1_1, W2_1, b2_1,
           Wc1, bc1, Wc2, bc2):
    del batch
    src = edge_index[0].astype(jnp.int32).reshape(NS, NCHUNK, K)
    dst = edge_index[1].astype(jnp.int32).reshape(NS, NCHUNK, K)
    zeros = jnp.zeros((ROWS_PER_TILE, DH), jnp.float32)

    wc2p = jnp.zeros((D, D), jnp.float32).at[:, :NUM_CLASSES].set(Wc2)
    bc2p = jnp.zeros((1, D), jnp.float32).at[0, :NUM_CLASSES].set(bc2)

    sc_aggregate = _build_sc_aggregate()
    xh = x.reshape(N_NODES, NC, DH).transpose(1, 0, 2)
    agg = sc_aggregate(xh, src, dst, zeros)
    h1, h1h = _tc_mlp(x, agg[0], agg[1], W1_0, b1_0, W2_0, b2_0)
    agg = sc_aggregate(h1h, src, dst, zeros)
    logits_p = _tc_head(h1, agg[0], agg[1], W1_1, b1_1, W2_1, b2_1,
                        Wc1, bc1, wc2p, bc2p)
    logits = logits_p[:, :NUM_CLASSES]
    return (logits, jnp.float32(0.0), jnp.float32(0.0))
